# Initial kernel scaffold; baseline (speedup 1.0000x reference)
import jax
import jax.numpy as jnp
from jax.experimental import pallas as pl


def kernel(inputs, token_table, pos_table):
    raise NotImplementedError("write your pallas kernel here")


# speedup vs baseline: 4.2564x; 4.2564x over previous
import functools

import jax
import jax.numpy as jnp
from jax import lax
from jax.experimental import pallas as pl
from jax.experimental.pallas import tpu as pltpu
from jax.experimental.pallas import tpu_sc as plsc

SEQ = 200
DIM = 128
BATCH = 4096
NC = 2
NS = 16
NW = NC * NS
ROWS = BATCH * SEQ
ROWS_PER_W = ROWS // NW
SEQ_PER_W = ROWS_PER_W // SEQ


def _emb_body(idx_hbm, tok_hbm, pos_hbm, out_hbm, idx_v, rows_v, pos_v, sem):
    wid = lax.axis_index("s") * NC + lax.axis_index("c")
    base = wid * ROWS_PER_W
    pltpu.sync_copy(pos_hbm, pos_v)

    def chunk_body(c, carry):
        off = base + c * SEQ
        pltpu.sync_copy(idx_hbm.at[pl.ds(off, SEQ)], idx_v)
        pltpu.async_copy(tok_hbm.at[idx_v], rows_v, sem).wait()

        def p_body(p, carry2):
            for d in range(DIM // 16):
                sl = pl.ds(d * 16, 16)
                rows_v[p, sl] = rows_v[p, sl] + pos_v[p, sl]
            return carry2

        lax.fori_loop(0, SEQ, p_body, 0, unroll=False)
        pltpu.sync_copy(rows_v, out_hbm.at[pl.ds(off, SEQ)])
        return carry

    lax.fori_loop(0, SEQ_PER_W, chunk_body, 0, unroll=False)


@functools.partial(jax.jit, static_argnums=())
def kernel(inputs, token_table, pos_table):
    idx_flat = inputs.reshape(ROWS).astype(jnp.int32)
    mesh = plsc.VectorSubcoreMesh(core_axis_name="c", subcore_axis_name="s")
    k = functools.partial(
        pl.kernel,
        out_type=jax.ShapeDtypeStruct((ROWS, DIM), jnp.float32),
        mesh=mesh,
        scratch_types=[
            pltpu.VMEM((SEQ,), jnp.int32),
            pltpu.VMEM((SEQ, DIM), jnp.float32),
            pltpu.VMEM((SEQ, DIM), jnp.float32),
            pltpu.SemaphoreType.DMA,
        ],
    )(_emb_body)
    out = k(idx_flat, token_table, pos_table)
    return out.reshape(BATCH, SEQ, DIM)
